# baseline (speedup 1.0000x reference)
import functools

import jax
import jax.numpy as jnp
from jax import lax
from jax.experimental import pallas as pl
from jax.experimental.pallas import tpu as pltpu
from jax.experimental.pallas import tpu_sc as plsc

D = 128
H = 50
B = 4096
N_ROWS = 100000


_R_BLK = 2048


def _scores_body(wa_ref, wb_ref, bias_ref, u_ref, i_ref, o_ref):
    u = u_ref[...]
    it = i_ref[...]
    o_ref[...] = (
        jnp.dot(u, wa_ref[...], preferred_element_type=jnp.float32)
        + jnp.dot(it, wb_ref[...], preferred_element_type=jnp.float32)
        + bias_ref[...]
    )


def _scores_tc(user_table, item_table, wa, wb, bias_row):
    n_blk = (N_ROWS + _R_BLK - 1) // _R_BLK
    return pl.pallas_call(
        _scores_body,
        grid=(n_blk,),
        in_specs=[
            pl.BlockSpec((D, 8), lambda i: (0, 0)),
            pl.BlockSpec((D, 8), lambda i: (0, 0)),
            pl.BlockSpec((1, 8), lambda i: (0, 0)),
            pl.BlockSpec((_R_BLK, D), lambda i: (i, 0)),
            pl.BlockSpec((_R_BLK, D), lambda i: (i, 0)),
        ],
        out_specs=pl.BlockSpec((_R_BLK, 8), lambda i: (i, 0)),
        out_shape=jax.ShapeDtypeStruct((N_ROWS, 8), jnp.float32),
    )(wa, wb, bias_row, user_table, item_table)


_NC = 2
_NS = 16
_NW = _NC * _NS
_BPW = B // _NW
_NG = _BPW // 16


def _sc_body(uid_hbm, iid_hbm, pvt_hbm, byt_hbm, fvt_hbm,
             us_hbm, is_hbm, pvs_hbm, bys_hbm, fvs_hbm,
             out_hbm,
             uidx, iidx, pvidx, byidx, fvidx,
             uval, ival, pvval, byval, fvval, obuf, sem):
    wid = lax.axis_index("s") * _NC + lax.axis_index("c")
    base = wid * _BPW
    hbase = wid * _BPW * H
    pltpu.sync_copy(uid_hbm.at[pl.ds(base, _BPW)], uidx)
    pltpu.sync_copy(iid_hbm.at[pl.ds(base, _BPW)], iidx)
    pltpu.sync_copy(pvt_hbm.at[pl.ds(hbase, _BPW * H)], pvidx)
    pltpu.sync_copy(byt_hbm.at[pl.ds(hbase, _BPW * H)], byidx)
    pltpu.sync_copy(fvt_hbm.at[pl.ds(hbase, _BPW * H)], fvidx)
    c0 = pltpu.async_copy(us_hbm.at[uidx], uval, sem)
    c1 = pltpu.async_copy(is_hbm.at[iidx], ival, sem)
    c2 = pltpu.async_copy(pvs_hbm.at[pvidx], pvval, sem)
    c3 = pltpu.async_copy(bys_hbm.at[byidx], byval, sem)
    c4 = pltpu.async_copy(fvs_hbm.at[fvidx], fvval, sem)
    c0.wait(); c1.wait(); c2.wait(); c3.wait(); c4.wait()
    inv_h = jnp.float32(1.0 / H)
    for g in range(_NG):
        sl = pl.ds(g * 16, 16)

        def hbody(h, acc):
            hsl = pl.ds(h * _BPW + g * 16, 16)
            return acc + pvval[hsl] + byval[hsl] + fvval[hsl]

        acc = lax.fori_loop(0, H, hbody, jnp.zeros((16,), jnp.float32))
        x = uval[sl] + ival[sl] + acc * inv_h
        obuf[sl] = 1.0 / (1.0 + jnp.exp(-x))
    pltpu.sync_copy(obuf, out_hbm.at[pl.ds(base, _BPW)])


def _sc_pool(user_id, item_id, pvt, byt, fvt, us, is_, pvs, bys, fvs):
    mesh = plsc.VectorSubcoreMesh(core_axis_name="c", subcore_axis_name="s",
                                  num_cores=_NC, num_subcores=_NS)
    run = pl.kernel(
        _sc_body,
        jax.ShapeDtypeStruct((B,), jnp.float32),
        mesh=mesh,
        scratch_types=[
            pltpu.VMEM((_BPW,), jnp.int32),
            pltpu.VMEM((_BPW,), jnp.int32),
            pltpu.VMEM((H * _BPW,), jnp.int32),
            pltpu.VMEM((H * _BPW,), jnp.int32),
            pltpu.VMEM((H * _BPW,), jnp.int32),
            pltpu.VMEM((_BPW,), jnp.float32),
            pltpu.VMEM((_BPW,), jnp.float32),
            pltpu.VMEM((H * _BPW,), jnp.float32),
            pltpu.VMEM((H * _BPW,), jnp.float32),
            pltpu.VMEM((H * _BPW,), jnp.float32),
            pltpu.VMEM((_BPW,), jnp.float32),
            pltpu.SemaphoreType.DMA,
        ],
    )
    return run(user_id, item_id, pvt, byt, fvt, us, is_, pvs, bys, fvs)


def kernel(user_id, item_id, pv_history, buy_history, fav_history,
           user_table, item_table, fc_w, fc_b):
    w = fc_w[0]
    zcol = jnp.zeros((D, 1), jnp.float32)
    wa = jnp.concatenate([w[0:128][:, None]] + [zcol] * 7, axis=1)
    wb = jnp.concatenate(
        [zcol, w[128:256][:, None], w[256:384][:, None],
         w[384:512][:, None], w[512:640][:, None], zcol, zcol, zcol], axis=1)
    bias_row = jnp.zeros((1, 8), jnp.float32).at[0, 0].set(fc_b[0])
    scores8 = _scores_tc(user_table, item_table, wa, wb, bias_row)
    us = scores8[:, 0]
    is_ = scores8[:, 1]
    pvs = scores8[:, 2]
    bys = scores8[:, 3]
    fvs = scores8[:, 4]

    def _rearrange(hist):
        return hist.reshape(_NW, _BPW, H).swapaxes(1, 2).reshape(-1)

    pvt = _rearrange(pv_history)
    byt = _rearrange(buy_history)
    fvt = _rearrange(fav_history)
    return _sc_pool(user_id, item_id, pvt, byt, fvt, us, is_, pvs, bys, fvs)


# speedup vs baseline: 2.4373x; 2.4373x over previous
import jax
import jax.numpy as jnp
from jax import lax
from jax.experimental import pallas as pl
from jax.experimental.pallas import tpu as pltpu
from jax.experimental.pallas import tpu_sc as plsc

D = 128
H = 50
B = 4096
N_ROWS = 100000


_R_BLK = 4096


def _scores_body(wu_ref, wi_ref, bias_ref, u_ref, i_ref,
                 o0_ref, o1_ref, o2_ref, o3_ref, o4_ref):
    u = u_ref[...]
    it = i_ref[...]
    dn = (((1,), (1,)), ((), ()))
    res = (
        lax.dot_general(wu_ref[...], u, dn, preferred_element_type=jnp.float32)
        + lax.dot_general(wi_ref[...], it, dn, preferred_element_type=jnp.float32)
    )
    b = bias_ref[0, 0]
    o0_ref[...] = res[0, :] + b
    o1_ref[...] = res[1, :]
    o2_ref[...] = res[2, :]
    o3_ref[...] = res[3, :]
    o4_ref[...] = res[4, :]


def _scores_tc(user_table, item_table, wu, wi, bias_row):
    n_blk = (N_ROWS + _R_BLK - 1) // _R_BLK
    one_d = pl.BlockSpec((_R_BLK,), lambda i: (i,))
    return pl.pallas_call(
        _scores_body,
        grid=(n_blk,),
        in_specs=[
            pl.BlockSpec((8, D), lambda i: (0, 0)),
            pl.BlockSpec((8, D), lambda i: (0, 0)),
            pl.BlockSpec((1, 8), lambda i: (0, 0)),
            pl.BlockSpec((_R_BLK, D), lambda i: (i, 0)),
            pl.BlockSpec((_R_BLK, D), lambda i: (i, 0)),
        ],
        out_specs=[one_d] * 5,
        out_shape=[jax.ShapeDtypeStruct((N_ROWS,), jnp.float32)] * 5,
    )(wu, wi, bias_row, user_table, item_table)


_NC = 2
_NS = 16
_NW = _NC * _NS
_BPW = B // _NW
_NG = _BPW // 16


def _sc_body(uid_hbm, iid_hbm, pvf_hbm, byf_hbm, fvf_hbm,
             us_hbm, is_hbm, pvs_hbm, bys_hbm, fvs_hbm,
             out_hbm,
             uidx, iidx, pvidx, byidx, fvidx,
             uval, ival, pvval, byval, fvval, obuf, sem):
    wid = lax.axis_index("s") * _NC + lax.axis_index("c")
    base = wid * _BPW
    hbase = base * H
    pltpu.sync_copy(uid_hbm.at[pl.ds(base, _BPW)], uidx)
    pltpu.sync_copy(iid_hbm.at[pl.ds(base, _BPW)], iidx)
    pltpu.sync_copy(pvf_hbm.at[pl.ds(hbase, _BPW * H)], pvidx)
    pltpu.sync_copy(byf_hbm.at[pl.ds(hbase, _BPW * H)], byidx)
    pltpu.sync_copy(fvf_hbm.at[pl.ds(hbase, _BPW * H)], fvidx)
    c0 = pltpu.async_copy(us_hbm.at[uidx], uval, sem)
    c1 = pltpu.async_copy(is_hbm.at[iidx], ival, sem)
    c2 = pltpu.async_copy(pvs_hbm.at[pvidx], pvval, sem)
    c3 = pltpu.async_copy(bys_hbm.at[byidx], byval, sem)
    c4 = pltpu.async_copy(fvs_hbm.at[fvidx], fvval, sem)
    c0.wait(); c1.wait(); c2.wait(); c3.wait(); c4.wait()
    inv_h = jnp.float32(1.0 / H)
    lanes = lax.iota(jnp.int32, 16)
    for g in range(_NG):
        sl = pl.ds(g * 16, 16)
        bvec = (g * 16 + lanes) * H

        def hbody(h, acc):
            idx = bvec + h
            return (acc
                    + plsc.load_gather(pvval, [idx])
                    + plsc.load_gather(byval, [idx])
                    + plsc.load_gather(fvval, [idx]))

        acc = lax.fori_loop(0, H, hbody, jnp.zeros((16,), jnp.float32))
        x = uval[sl] + ival[sl] + acc * inv_h
        obuf[sl] = 1.0 / (1.0 + jnp.exp(-x))
    pltpu.sync_copy(obuf, out_hbm.at[pl.ds(base, _BPW)])


def _sc_pool(user_id, item_id, pvf, byf, fvf, us, is_, pvs, bys, fvs):
    mesh = plsc.VectorSubcoreMesh(core_axis_name="c", subcore_axis_name="s",
                                  num_cores=_NC, num_subcores=_NS)
    run = pl.kernel(
        _sc_body,
        jax.ShapeDtypeStruct((B,), jnp.float32),
        mesh=mesh,
        compiler_params=pltpu.CompilerParams(needs_layout_passes=False),
        scratch_types=[
            pltpu.VMEM((_BPW,), jnp.int32),
            pltpu.VMEM((_BPW,), jnp.int32),
            pltpu.VMEM((H * _BPW,), jnp.int32),
            pltpu.VMEM((H * _BPW,), jnp.int32),
            pltpu.VMEM((H * _BPW,), jnp.int32),
            pltpu.VMEM((_BPW,), jnp.float32),
            pltpu.VMEM((_BPW,), jnp.float32),
            pltpu.VMEM((H * _BPW,), jnp.float32),
            pltpu.VMEM((H * _BPW,), jnp.float32),
            pltpu.VMEM((H * _BPW,), jnp.float32),
            pltpu.VMEM((_BPW,), jnp.float32),
            pltpu.SemaphoreType.DMA,
        ],
    )
    return run(user_id, item_id, pvf, byf, fvf, us, is_, pvs, bys, fvs)


def kernel(user_id, item_id, pv_history, buy_history, fav_history,
           user_table, item_table, fc_w, fc_b):
    w = fc_w[0]
    zrow = jnp.zeros((1, D), jnp.float32)
    wu = jnp.concatenate([w[0:128][None, :]] + [zrow] * 7, axis=0)
    wi = jnp.concatenate(
        [zrow, w[128:256][None, :], w[256:384][None, :],
         w[384:512][None, :], w[512:640][None, :], zrow, zrow, zrow], axis=0)
    bias_row = jnp.zeros((1, 8), jnp.float32).at[0, 0].set(fc_b[0])
    us, is_, pvs, bys, fvs = _scores_tc(user_table, item_table, wu, wi, bias_row)
    return _sc_pool(user_id, item_id,
                    pv_history.reshape(-1), buy_history.reshape(-1),
                    fav_history.reshape(-1), us, is_, pvs, bys, fvs)


# speedup vs baseline: 2.5849x; 1.0606x over previous
import jax
import jax.numpy as jnp
from jax import lax
from jax.experimental import pallas as pl
from jax.experimental.pallas import tpu as pltpu
from jax.experimental.pallas import tpu_sc as plsc

D = 128
H = 50
B = 4096
N_ROWS = 100000


_R_BLK = 8192


def _scores_body(wu_ref, wi_ref, bias_ref, u_ref, i_ref,
                 o0_ref, o1_ref, o2_ref, o3_ref, o4_ref):
    u = u_ref[...]
    it = i_ref[...]
    dn = (((1,), (1,)), ((), ()))
    res = (
        lax.dot_general(wu_ref[...], u, dn, preferred_element_type=jnp.float32)
        + lax.dot_general(wi_ref[...], it, dn, preferred_element_type=jnp.float32)
    )
    b = bias_ref[0, 0]
    o0_ref[...] = res[0, :] + b
    o1_ref[...] = res[1, :]
    o2_ref[...] = res[2, :]
    o3_ref[...] = res[3, :]
    o4_ref[...] = res[4, :]


def _scores_tc(user_table, item_table, wu, wi, bias_row):
    n_blk = (N_ROWS + _R_BLK - 1) // _R_BLK
    one_d = pl.BlockSpec((_R_BLK,), lambda i: (i,))
    return pl.pallas_call(
        _scores_body,
        grid=(n_blk,),
        in_specs=[
            pl.BlockSpec((8, D), lambda i: (0, 0)),
            pl.BlockSpec((8, D), lambda i: (0, 0)),
            pl.BlockSpec((1, 8), lambda i: (0, 0)),
            pl.BlockSpec((_R_BLK, D), lambda i: (i, 0)),
            pl.BlockSpec((_R_BLK, D), lambda i: (i, 0)),
        ],
        out_specs=[one_d] * 5,
        out_shape=[jax.ShapeDtypeStruct((N_ROWS,), jnp.float32)] * 5,
    )(wu, wi, bias_row, user_table, item_table)


_NC = 2
_NS = 16
_NW = _NC * _NS
_BPW = B // _NW
_NG = _BPW // 16


def _sc_body(uid_hbm, iid_hbm, pvf_hbm, byf_hbm, fvf_hbm,
             us_hbm, is_hbm, pvs_hbm, bys_hbm, fvs_hbm,
             out_hbm,
             uidx, iidx, pvidx, byidx, fvidx,
             uval, ival, pvval, byval, fvval, obuf, sem):
    wid = lax.axis_index("s") * _NC + lax.axis_index("c")
    base = wid * _BPW
    hbase = base * H
    pltpu.sync_copy(uid_hbm.at[pl.ds(base, _BPW)], uidx)
    pltpu.sync_copy(iid_hbm.at[pl.ds(base, _BPW)], iidx)
    pltpu.sync_copy(pvf_hbm.at[pl.ds(hbase, _BPW * H)], pvidx)
    pltpu.sync_copy(byf_hbm.at[pl.ds(hbase, _BPW * H)], byidx)
    pltpu.sync_copy(fvf_hbm.at[pl.ds(hbase, _BPW * H)], fvidx)
    c0 = pltpu.async_copy(us_hbm.at[uidx], uval, sem)
    c1 = pltpu.async_copy(is_hbm.at[iidx], ival, sem)
    c2 = pltpu.async_copy(pvs_hbm.at[pvidx], pvval, sem)
    c3 = pltpu.async_copy(bys_hbm.at[byidx], byval, sem)
    c4 = pltpu.async_copy(fvs_hbm.at[fvidx], fvval, sem)
    c0.wait(); c1.wait(); c2.wait(); c3.wait(); c4.wait()
    inv_h = jnp.float32(1.0 / H)
    lanes = lax.iota(jnp.int32, 16)
    for g in range(_NG):
        sl = pl.ds(g * 16, 16)
        bvec = (g * 16 + lanes) * H

        def hbody(h, acc):
            idx = bvec + h
            return (acc
                    + plsc.load_gather(pvval, [idx])
                    + plsc.load_gather(byval, [idx])
                    + plsc.load_gather(fvval, [idx]))

        acc = lax.fori_loop(0, H, hbody, jnp.zeros((16,), jnp.float32))
        x = uval[sl] + ival[sl] + acc * inv_h
        obuf[sl] = 1.0 / (1.0 + jnp.exp(-x))
    pltpu.sync_copy(obuf, out_hbm.at[pl.ds(base, _BPW)])


def _sc_pool(user_id, item_id, pvf, byf, fvf, us, is_, pvs, bys, fvs):
    mesh = plsc.VectorSubcoreMesh(core_axis_name="c", subcore_axis_name="s",
                                  num_cores=_NC, num_subcores=_NS)
    run = pl.kernel(
        _sc_body,
        jax.ShapeDtypeStruct((B,), jnp.float32),
        mesh=mesh,
        compiler_params=pltpu.CompilerParams(needs_layout_passes=False),
        scratch_types=[
            pltpu.VMEM((_BPW,), jnp.int32),
            pltpu.VMEM((_BPW,), jnp.int32),
            pltpu.VMEM((H * _BPW,), jnp.int32),
            pltpu.VMEM((H * _BPW,), jnp.int32),
            pltpu.VMEM((H * _BPW,), jnp.int32),
            pltpu.VMEM((_BPW,), jnp.float32),
            pltpu.VMEM((_BPW,), jnp.float32),
            pltpu.VMEM((H * _BPW,), jnp.float32),
            pltpu.VMEM((H * _BPW,), jnp.float32),
            pltpu.VMEM((H * _BPW,), jnp.float32),
            pltpu.VMEM((_BPW,), jnp.float32),
            pltpu.SemaphoreType.DMA,
        ],
    )
    return run(user_id, item_id, pvf, byf, fvf, us, is_, pvs, bys, fvs)


def kernel(user_id, item_id, pv_history, buy_history, fav_history,
           user_table, item_table, fc_w, fc_b):
    w = fc_w[0]
    zrow = jnp.zeros((1, D), jnp.float32)
    wu = jnp.concatenate([w[0:128][None, :]] + [zrow] * 7, axis=0)
    wi = jnp.concatenate(
        [zrow, w[128:256][None, :], w[256:384][None, :],
         w[384:512][None, :], w[512:640][None, :], zrow, zrow, zrow], axis=0)
    bias_row = jnp.zeros((1, 8), jnp.float32).at[0, 0].set(fc_b[0])
    us, is_, pvs, bys, fvs = _scores_tc(user_table, item_table, wu, wi, bias_row)
    return _sc_pool(user_id, item_id,
                    pv_history.reshape(-1), buy_history.reshape(-1),
                    fav_history.reshape(-1), us, is_, pvs, bys, fvs)
